# baseline (speedup 1.0000x reference)
import functools

import jax
import jax.numpy as jnp
from jax.experimental import pallas as pl

_MARGIN = 1.0


def _triplet_block(x_blk_ref, x_all_ref, y_all_ref, out_ref, *, br):
    i = pl.program_id(0)

    xb = x_blk_ref[...]
    xf = x_all_ref[...]
    y_all = y_all_ref[0, :]
    yb = y_all_ref[0, pl.ds(i * br, br)]

    g = jax.lax.dot_general(
        xb, xf,
        dimension_numbers=(((1,), (1,)), ((), ())),
        preferred_element_type=jnp.float32,
    )
    x2b = jnp.sum(xb * xb, axis=1)
    x2f = jnp.sum(xf * xf, axis=1)
    d2 = x2b[:, None] + x2f[None, :] - 2.0 * g

    mask = yb[:, None] == y_all[None, :]
    dist_pos = jnp.max(jnp.where(mask, d2, -jnp.inf), axis=1)
    dist_neg = jnp.min(jnp.where(mask, jnp.inf, d2), axis=1)
    partial = jnp.sum(jax.nn.relu(dist_pos + _MARGIN - dist_neg)).reshape(1, 1)

    @pl.when(i == 0)
    def _init():
        out_ref[...] = jnp.zeros((1, 1), jnp.float32)

    out_ref[...] += partial


def kernel(x, y):
    n, d = x.shape
    br = 512
    grid = n // br
    y2d = y.reshape(1, n)

    out = pl.pallas_call(
        functools.partial(_triplet_block, br=br),
        grid=(grid,),
        in_specs=[
            pl.BlockSpec((br, d), lambda i: (i, 0)),
            pl.BlockSpec((n, d), lambda i: (0, 0)),
            pl.BlockSpec((1, n), lambda i: (0, 0)),
        ],
        out_specs=pl.BlockSpec((1, 1), lambda i: (0, 0)),
        out_shape=jax.ShapeDtypeStruct((1, 1), jnp.float32),
    )(x, x, y2d)
    return out[0, 0] / n
